# Initial kernel scaffold; baseline (speedup 1.0000x reference)
import jax
import jax.numpy as jnp
from jax.experimental import pallas as pl


def kernel(hidden_states, position_ids, ln1_w, Wqkv, q_ln_w, k_ln_w, Wo, ln2_w, Wr, Wg, Wu, Wd, Wsg, Wsu, Wsd):
    raise NotImplementedError("write your pallas kernel here")


# speedup vs baseline: 1.2193x; 1.2193x over previous
import jax
import jax.numpy as jnp
from jax.experimental import pallas as pl

_S, _D = 2048, 1024
_H, _HKV, _HD = 16, 4, 64
_E, _DFF = 8, 512
_G = _H // _HKV
_QKVD = (_H + 2 * _HKV) * _HD
_EPS = 1e-6
_THETA = 10000.0
_ST = 256
_QT = 512


def _stage1_kernel(pos_ref, inv_ref, x_ref, w1_ref, wqkv_ref,
                   qkv_ref, cos_ref, sin_ref):
    x = x_ref[...]
    v = jnp.mean(x * x, axis=-1, keepdims=True)
    xn = w1_ref[...] * (x * jax.lax.rsqrt(v + _EPS))
    qkv_ref[...] = jax.lax.dot_general(
        xn.astype(jnp.bfloat16), wqkv_ref[...],
        (((1,), (1,)), ((), ())), preferred_element_type=jnp.float32)
    f = pos_ref[...] * inv_ref[...]
    cos_ref[...] = jnp.cos(f)
    sin_ref[...] = jnp.sin(f)


def _rope(xn, c, s):
    x1 = xn[:, : _HD // 2]
    x2 = xn[:, _HD // 2:]
    return jnp.concatenate([x1 * c - x2 * s, x2 * c + x1 * s], axis=-1)


def _attn_kernel(cos_ref, sin_ref, qln_ref, kln_ref, q_ref, kv_ref,
                 ctx_ref):
    t = pl.program_id(0)
    h2 = pl.program_id(1)
    g2 = h2 // 2
    cf = cos_ref[...]
    sf = sin_ref[...]
    cq = cos_ref[pl.ds(t * _QT, _QT), :]
    sq = sin_ref[pl.ds(t * _QT, _QT), :]

    kv = kv_ref[...]
    ksel = jnp.where(
        g2 < 2,
        jnp.where(g2 == 0, kv[:, 0:_HD], kv[:, _HD:2 * _HD]),
        jnp.where(g2 == 2, kv[:, 2 * _HD:3 * _HD], kv[:, 3 * _HD:4 * _HD]))
    vsel = jnp.where(
        g2 < 2,
        jnp.where(g2 == 0, kv[:, 4 * _HD:5 * _HD], kv[:, 5 * _HD:6 * _HD]),
        jnp.where(g2 == 2, kv[:, 6 * _HD:7 * _HD], kv[:, 7 * _HD:8 * _HD]))
    kvar = jnp.mean(ksel * ksel, axis=-1, keepdims=True)
    kn = kln_ref[...] * (ksel * jax.lax.rsqrt(kvar + _EPS))
    kr = _rope(kn, cf, sf).astype(jnp.bfloat16)
    vb = vsel.astype(jnp.bfloat16)

    qpair = q_ref[...]

    def head(q):
        qv = jnp.mean(q * q, axis=-1, keepdims=True)
        qn = qln_ref[...] * (q * jax.lax.rsqrt(qv + _EPS))
        qr = _rope(qn, cq, sq).astype(jnp.bfloat16)
        scores = jax.lax.dot_general(
            qr, kr, (((1,), (1,)), ((), ())),
            preferred_element_type=jnp.float32) * (1.0 / 8.0)
        m = jnp.max(scores, axis=-1, keepdims=True)
        p = jnp.exp(scores - m)
        attn = (p / jnp.sum(p, axis=-1, keepdims=True)).astype(jnp.bfloat16)
        return jax.lax.dot_general(
            attn, vb, (((1,), (0,)), ((), ())),
            preferred_element_type=jnp.float32)

    ctx_ref[...] = jnp.concatenate(
        [head(qpair[:, :_HD]), head(qpair[:, _HD:])], axis=-1)


def _stage3_kernel(res_ref, ctx_ref, wo_ref, ln2_ref, wr_ref,
                   h_ref, x2b_ref, we_ref):
    ao = jax.lax.dot_general(
        ctx_ref[...].astype(jnp.bfloat16), wo_ref[...],
        (((1,), (1,)), ((), ())), preferred_element_type=jnp.float32)
    h = res_ref[...] + ao
    h_ref[...] = h
    v = jnp.mean(h * h, axis=-1, keepdims=True)
    x2 = ln2_ref[...] * (h * jax.lax.rsqrt(v + _EPS))
    x2b_ref[...] = x2.astype(jnp.bfloat16)
    logits = jax.lax.dot_general(
        x2, wr_ref[...], (((1,), (1,)), ((), ())),
        preferred_element_type=jnp.float32)
    lm = jnp.max(logits, axis=-1, keepdims=True)
    el = jnp.exp(logits - lm)
    p = el / jnp.sum(el, axis=-1, keepdims=True)
    it = jax.lax.broadcasted_iota(jnp.int32, p.shape, 1)
    m1 = jnp.max(p, axis=-1, keepdims=True)
    i1 = jnp.min(jnp.where(p == m1, it, _E), axis=-1, keepdims=True)
    p2 = jnp.where(it == i1, -1.0, p)
    m2 = jnp.max(p2, axis=-1, keepdims=True)
    i2 = jnp.min(jnp.where(p2 == m2, it, _E), axis=-1, keepdims=True)
    we = jnp.where(it == i1, m1, 0.0) + jnp.where(it == i2, m2, 0.0)
    we_ref[...] = we / (m1 + m2)


def _moe_kernel(x2b_ref, h_ref, we_ref, wg_ref, wu_ref, wd_ref, out_ref):
    c = pl.program_id(0)
    x2b = x2b_ref[...]
    g = jax.lax.dot_general(
        x2b, wg_ref[0], (((1,), (1,)), ((), ())),
        preferred_element_type=jnp.float32)
    u = jax.lax.dot_general(
        x2b, wu_ref[0], (((1,), (1,)), ((), ())),
        preferred_element_type=jnp.float32)
    a = (g * jax.nn.sigmoid(g)) * u
    it = jax.lax.broadcasted_iota(jnp.int32, we_ref.shape, 1)
    w = jnp.sum(jnp.where(it == jnp.minimum(c, _E - 1), we_ref[...], 0.0),
                axis=-1, keepdims=True)
    a = a * jnp.where(c == _E, 1.0, w)
    part = jax.lax.dot_general(
        a.astype(jnp.bfloat16), wd_ref[0], (((1,), (1,)), ((), ())),
        preferred_element_type=jnp.float32)

    @pl.when(c == 0)
    def _():
        out_ref[...] = h_ref[...]

    out_ref[...] += part


def kernel(hidden_states, position_ids, ln1_w, Wqkv, q_ln_w, k_ln_w, Wo,
           ln2_w, Wr, Wg, Wu, Wd, Wsg, Wsu, Wsd):
    x = hidden_states.reshape(_S, _D)
    pos = position_ids.reshape(_S, 1).astype(jnp.float32)
    inv_freq = (1.0 / (_THETA ** (jnp.arange(0, _HD, 2, dtype=jnp.float32)
                                  / _HD))).reshape(1, _HD // 2)
    ln1 = ln1_w.reshape(1, _D)
    ln2 = ln2_w.reshape(1, _D)
    qln = q_ln_w.reshape(1, _HD)
    kln = k_ln_w.reshape(1, _HD)
    wqkv_bf = Wqkv.astype(jnp.bfloat16)
    wo_bf = Wo.astype(jnp.bfloat16)

    nt = _S // _ST
    qkv, cos, sin = pl.pallas_call(
        _stage1_kernel,
        grid=(nt,),
        in_specs=[
            pl.BlockSpec((_ST, 1), lambda i: (i, 0)),
            pl.BlockSpec((1, _HD // 2), lambda i: (0, 0)),
            pl.BlockSpec((_ST, _D), lambda i: (i, 0)),
            pl.BlockSpec((1, _D), lambda i: (0, 0)),
            pl.BlockSpec((_QKVD, _D), lambda i: (0, 0)),
        ],
        out_specs=[
            pl.BlockSpec((_ST, _QKVD), lambda i: (i, 0)),
            pl.BlockSpec((_ST, _HD // 2), lambda i: (i, 0)),
            pl.BlockSpec((_ST, _HD // 2), lambda i: (i, 0)),
        ],
        out_shape=[
            jax.ShapeDtypeStruct((_S, _QKVD), jnp.float32),
            jax.ShapeDtypeStruct((_S, _HD // 2), jnp.float32),
            jax.ShapeDtypeStruct((_S, _HD // 2), jnp.float32),
        ],
    )(pos, inv_freq, x, ln1, wqkv_bf)

    ctx = pl.pallas_call(
        _attn_kernel,
        grid=(_S // _QT, _H // 2),
        in_specs=[
            pl.BlockSpec((_S, _HD // 2), lambda t, h2: (0, 0)),
            pl.BlockSpec((_S, _HD // 2), lambda t, h2: (0, 0)),
            pl.BlockSpec((1, _HD), lambda t, h2: (0, 0)),
            pl.BlockSpec((1, _HD), lambda t, h2: (0, 0)),
            pl.BlockSpec((_QT, 2 * _HD), lambda t, h2: (t, h2)),
            pl.BlockSpec((_S, 512), lambda t, h2: (0, 2)),
        ],
        out_specs=pl.BlockSpec((_QT, 2 * _HD), lambda t, h2: (t, h2)),
        out_shape=jax.ShapeDtypeStruct((_S, _H * _HD), jnp.float32),
    )(cos, sin, qln, kln, qkv, qkv)

    h, x2b, we = pl.pallas_call(
        _stage3_kernel,
        grid=(nt,),
        in_specs=[
            pl.BlockSpec((_ST, _D), lambda i: (i, 0)),
            pl.BlockSpec((_ST, _D), lambda i: (i, 0)),
            pl.BlockSpec((_D, _D), lambda i: (0, 0)),
            pl.BlockSpec((1, _D), lambda i: (0, 0)),
            pl.BlockSpec((_E, _D), lambda i: (0, 0)),
        ],
        out_specs=[
            pl.BlockSpec((_ST, _D), lambda i: (i, 0)),
            pl.BlockSpec((_ST, _D), lambda i: (i, 0)),
            pl.BlockSpec((_ST, _E), lambda i: (i, 0)),
        ],
        out_shape=[
            jax.ShapeDtypeStruct((_S, _D), jnp.float32),
            jax.ShapeDtypeStruct((_S, _D), jnp.bfloat16),
            jax.ShapeDtypeStruct((_S, _E), jnp.float32),
        ],
    )(x, ctx, wo_bf, ln2, Wr)

    wg_cat = jnp.concatenate(
        [Wg, Wsg.reshape(1, _DFF, _D)], axis=0).astype(jnp.bfloat16)
    wu_cat = jnp.concatenate(
        [Wu, Wsu.reshape(1, _DFF, _D)], axis=0).astype(jnp.bfloat16)
    wd_cat = jnp.concatenate(
        [Wd, Wsd.reshape(1, _D, _DFF)], axis=0).astype(jnp.bfloat16)

    out = pl.pallas_call(
        _moe_kernel,
        grid=(_E + 1,),
        in_specs=[
            pl.BlockSpec((_S, _D), lambda c: (0, 0)),
            pl.BlockSpec((_S, _D), lambda c: (0, 0)),
            pl.BlockSpec((_S, _E), lambda c: (0, 0)),
            pl.BlockSpec((1, _DFF, _D), lambda c: (c, 0, 0)),
            pl.BlockSpec((1, _DFF, _D), lambda c: (c, 0, 0)),
            pl.BlockSpec((1, _D, _DFF), lambda c: (c, 0, 0)),
        ],
        out_specs=pl.BlockSpec((_S, _D), lambda c: (0, 0)),
        out_shape=jax.ShapeDtypeStruct((_S, _D), jnp.float32),
    )(x2b, h, we, wg_cat, wu_cat, wd_cat)

    return out.reshape(1, _S, _D)
